# baseline (speedup 1.0000x reference)
import functools

import jax
import jax.numpy as jnp
from jax import lax
from jax.experimental import pallas as pl
from jax.experimental.pallas import tpu as pltpu
from jax.experimental.pallas import tpu_sc as plsc

_C = 100000
_IGNORE = 0
_SMOOTH = 0.1
_CONF = 1.0 - _SMOOTH
_EPS = _SMOOTH / (_C - 1)

_BR = 64
_R_SC = 512
_NC = 2
_NS = 16
_NW = _NC * _NS
_RPW = _R_SC // _NW
_CW = 2560
_C_SC = 99840
_NCH = _C_SC // _CW
_TAILW = 256
_UNROLL = 10


def _tc_body(tgt_ref, logit_ref, ttgt_ref, tail_ref, out_ref):
    i = pl.program_id(0)
    t = tgt_ref[...]
    col = jax.lax.broadcasted_iota(jnp.int32, (_BR, _C), 1)
    blk = logit_ref[...]
    s = jnp.sum(blk, axis=1, keepdims=True)
    g = jnp.sum(jnp.where(col == t, blk, 0.0), axis=1, keepdims=True)
    per_row = _EPS * s + (_CONF - _EPS) * g
    partial = jnp.sum(jnp.where(t != _IGNORE, per_row, 0.0))

    tt = ttgt_ref[...]
    tcol = jax.lax.broadcasted_iota(jnp.int32, (_R_SC, _TAILW), 1) + _C_SC
    tblk = tail_ref[...]
    ts = jnp.sum(tblk, axis=1, keepdims=True)
    tg = jnp.sum(jnp.where(tcol == tt, tblk, 0.0), axis=1, keepdims=True)
    tper = _EPS * ts + (_CONF - _EPS) * tg
    tpartial = jnp.sum(jnp.where(tt != _IGNORE, tper, 0.0))

    partial = partial + lax.select(i == 0, tpartial, 0.0)
    out_ref[...] = jnp.full((1, 1, 1), partial, jnp.float32)


def _tc_part(logit, tgt, tail, tail_tgt, r_tc):
    nblk = r_tc // _BR
    out = pl.pallas_call(
        _tc_body,
        grid=(nblk,),
        in_specs=[
            pl.BlockSpec((_BR, 1), lambda i: (i, 0)),
            pl.BlockSpec((_BR, _C), lambda i: (i, 0)),
            pl.BlockSpec((_R_SC, 1), lambda i: (0, 0)),
            pl.BlockSpec((_R_SC, _TAILW), lambda i: (0, 0)),
        ],
        out_specs=pl.BlockSpec((1, 1, 1), lambda i: (i, 0, 0)),
        out_shape=jax.ShapeDtypeStruct((nblk, 1, 1), jnp.float32),
        compiler_params=pltpu.CompilerParams(
            dimension_semantics=("arbitrary",),
        ),
    )(tgt, logit, tail_tgt, tail)
    return out


def _make_sc_kernel(r_tc):
    mesh = plsc.VectorSubcoreMesh(core_axis_name="c", subcore_axis_name="s")

    @functools.partial(
        pl.kernel,
        mesh=mesh,
        out_type=jax.ShapeDtypeStruct((_R_SC, 16), jnp.float32),
        scratch_types=[
            pltpu.VMEM((16,), jnp.int32),
            pltpu.VMEM((_RPW, _CW), jnp.float32),
            pltpu.VMEM((_RPW, _CW), jnp.float32),
            pltpu.VMEM((_RPW, 16), jnp.float32),
            pltpu.VMEM((_RPW, 16), jnp.float32),
            pltpu.VMEM((_RPW, 16), jnp.float32),
            pltpu.SemaphoreType.DMA,
            pltpu.SemaphoreType.DMA,
        ],
    )
    def _sc_kernel(logit_hbm, tgt_hbm, out_hbm, tgt_v, buf0, buf1,
                   s_rows, g_rows, rows_v, sem0, sem1):
        wid = lax.axis_index("s") * _NC + lax.axis_index("c")
        row_base = r_tc + wid * _RPW
        pltpu.sync_copy(tgt_hbm.at[pl.ds(row_base, _RPW)], tgt_v)
        for rr in range(_RPW):
            s_rows[rr, :] = jnp.zeros((16,), jnp.float32)
            g_rows[rr, :] = jnp.zeros((16,), jnp.float32)

        def start(ch, buf, sem):
            pltpu.make_async_copy(
                logit_hbm.at[pl.ds(row_base, _RPW), pl.ds(ch * _CW, _CW)],
                buf, sem).start()

        def wait(buf, sem):
            pltpu.make_async_copy(
                logit_hbm.at[pl.ds(row_base, _RPW), pl.ds(0, _CW)],
                buf, sem).wait()

        def proc(buf, ch):
            base = ch * _CW
            for rr in range(_RPW):
                t_spl = (tgt_v[...].at[jnp.full((16,), rr, jnp.int32)]
                         .get(mode="promise_in_bounds"))

                def step(k, carry, rr=rr, t_spl=t_spl):
                    s_a, g_a = carry
                    for u in range(_UNROLL):
                        off = (k * _UNROLL + u) * 16
                        vals = buf[rr, pl.ds(off, 16)]
                        gcol = lax.iota(jnp.int32, 16) + (base + off)
                        s_a = s_a + vals
                        g_a = g_a + jnp.where(gcol == t_spl, vals, 0.0)
                    return s_a, g_a

                s_a, g_a = lax.fori_loop(
                    0, _CW // (16 * _UNROLL), step,
                    (jnp.zeros((16,), jnp.float32),
                     jnp.zeros((16,), jnp.float32)))
                s_rows[rr, :] = s_rows[rr, :] + s_a
                g_rows[rr, :] = g_rows[rr, :] + g_a

        start(0, buf0, sem0)
        start(1, buf1, sem1)

        def ring(k, _):
            wait(buf0, sem0)
            proc(buf0, 2 * k)
            start(2 * k + 2, buf0, sem0)
            wait(buf1, sem1)
            proc(buf1, 2 * k + 1)
            start(2 * k + 3, buf1, sem1)
            return 0

        lax.fori_loop(0, (_NCH - 3) // 2, ring, 0)
        wait(buf0, sem0)
        proc(buf0, _NCH - 3)
        start(_NCH - 1, buf0, sem0)
        wait(buf1, sem1)
        proc(buf1, _NCH - 2)
        wait(buf0, sem0)
        proc(buf0, _NCH - 1)

        for rr in range(_RPW):
            t_spl = (tgt_v[...].at[jnp.full((16,), rr, jnp.int32)]
                     .get(mode="promise_in_bounds"))
            per_row = _EPS * s_rows[rr, :] + (_CONF - _EPS) * g_rows[rr, :]
            rows_v[rr, :] = per_row * jnp.minimum(t_spl, 1).astype(
                jnp.float32)
        pltpu.sync_copy(rows_v, out_hbm.at[pl.ds(wid * _RPW, _RPW)])

    return _sc_kernel


def kernel(logit, target):
    n = logit.shape[0]
    r_tc = n - _R_SC
    tgt = target.astype(jnp.int32)
    tail = jnp.pad(lax.slice(logit, (r_tc, _C_SC), (n, _C)),
                   ((0, 0), (0, _TAILW - (_C - _C_SC))))
    tc_out = _tc_part(logit, tgt.reshape(n, 1), tail,
                      tgt[r_tc:].reshape(_R_SC, 1), r_tc)
    sc_out = _make_sc_kernel(r_tc)(logit, tgt)
    return -(jnp.sum(tc_out) + jnp.sum(sc_out))


# speedup vs baseline: 1.0013x; 1.0013x over previous
import functools

import jax
import jax.numpy as jnp
from jax import lax
from jax.experimental import pallas as pl
from jax.experimental.pallas import tpu as pltpu
from jax.experimental.pallas import tpu_sc as plsc

_C = 100000
_IGNORE = 0
_SMOOTH = 0.1
_CONF = 1.0 - _SMOOTH
_EPS = _SMOOTH / (_C - 1)

_BR = 64
_R_SC = 512
_NC = 2
_NS = 16
_NW = _NC * _NS
_RPW = _R_SC // _NW
_CW = 2560
_C_SC = 99840
_NCH = _C_SC // _CW
_TAILW = 256
_UNROLL = 10


def _tc_body(tgt_ref, logit_ref, ttgt_ref, tail_ref, out_ref):
    i = pl.program_id(0)
    t = tgt_ref[...]
    col = jax.lax.broadcasted_iota(jnp.int32, (_BR, _C), 1)
    blk = logit_ref[...]
    s = jnp.sum(blk, axis=1, keepdims=True)
    g = jnp.sum(jnp.where(col == t, blk, 0.0), axis=1, keepdims=True)
    per_row = _EPS * s + (_CONF - _EPS) * g
    partial = jnp.sum(jnp.where(t != _IGNORE, per_row, 0.0))

    tt = ttgt_ref[...]
    tcol = jax.lax.broadcasted_iota(jnp.int32, (_R_SC, _TAILW), 1) + _C_SC
    tblk = tail_ref[...]
    ts = jnp.sum(tblk, axis=1, keepdims=True)
    tg = jnp.sum(jnp.where(tcol == tt, tblk, 0.0), axis=1, keepdims=True)
    tper = _EPS * ts + (_CONF - _EPS) * tg
    tpartial = jnp.sum(jnp.where(tt != _IGNORE, tper, 0.0))

    partial = partial + lax.select(i == 0, tpartial, 0.0)
    out_ref[...] = jnp.full((1, 1, 1), partial, jnp.float32)


def _tc_part(logit, tgt, tail, tail_tgt, r_tc):
    nblk = r_tc // _BR
    out = pl.pallas_call(
        _tc_body,
        grid=(nblk,),
        in_specs=[
            pl.BlockSpec((_BR, 1), lambda i: (i, 0)),
            pl.BlockSpec((_BR, _C), lambda i: (i, 0)),
            pl.BlockSpec((_R_SC, 1), lambda i: (0, 0)),
            pl.BlockSpec((_R_SC, _TAILW), lambda i: (0, 0)),
        ],
        out_specs=pl.BlockSpec((1, 1, 1), lambda i: (i, 0, 0)),
        out_shape=jax.ShapeDtypeStruct((nblk, 1, 1), jnp.float32),
        compiler_params=pltpu.CompilerParams(
            dimension_semantics=("arbitrary",),
        ),
    )(tgt, logit, tail_tgt, tail)
    return out


def _make_sc_kernel(r_tc):
    mesh = plsc.VectorSubcoreMesh(core_axis_name="c", subcore_axis_name="s")

    @functools.partial(
        pl.kernel,
        mesh=mesh,
        out_type=jax.ShapeDtypeStruct((_R_SC, 16), jnp.float32),
        scratch_types=[
            pltpu.VMEM((16,), jnp.int32),
            pltpu.VMEM((_RPW, _CW), jnp.float32),
            pltpu.VMEM((_RPW, _CW), jnp.float32),
            pltpu.VMEM((_RPW, 16), jnp.float32),
            pltpu.VMEM((_RPW, 16), jnp.float32),
            pltpu.VMEM((_RPW, 16), jnp.float32),
            pltpu.SemaphoreType.DMA,
            pltpu.SemaphoreType.DMA,
        ],
    )
    def _sc_kernel(logit_hbm, tgt_hbm, out_hbm, tgt_v, buf0, buf1,
                   s_rows, g_rows, rows_v, sem0, sem1):
        wid = lax.axis_index("s") * _NC + lax.axis_index("c")
        row_base = r_tc + wid * _RPW
        pltpu.sync_copy(tgt_hbm.at[pl.ds(row_base, _RPW)], tgt_v)
        for rr in range(_RPW):
            s_rows[rr, :] = jnp.zeros((16,), jnp.float32)
            g_rows[rr, :] = jnp.zeros((16,), jnp.float32)

        def start(ch, buf, sem):
            pltpu.make_async_copy(
                logit_hbm.at[pl.ds(row_base, _RPW), pl.ds(ch * _CW, _CW)],
                buf, sem).start()

        def wait(buf, sem):
            pltpu.make_async_copy(
                logit_hbm.at[pl.ds(row_base, _RPW), pl.ds(0, _CW)],
                buf, sem).wait()

        def proc(buf, ch):
            base = ch * _CW
            for rr in range(_RPW):
                t_spl = (tgt_v[...].at[jnp.full((16,), rr, jnp.int32)]
                         .get(mode="promise_in_bounds"))

                def step(k, carry, rr=rr, t_spl=t_spl):
                    s_a, g_a = carry
                    for u in range(_UNROLL):
                        off = (k * _UNROLL + u) * 16
                        vals = buf[rr, pl.ds(off, 16)]
                        gcol = lax.iota(jnp.int32, 16) + (base + off)
                        s_a = s_a + vals
                        g_a = g_a + jnp.where(gcol == t_spl, vals, 0.0)
                    return s_a, g_a

                s_a, g_a = lax.fori_loop(
                    0, _CW // (16 * _UNROLL), step,
                    (jnp.zeros((16,), jnp.float32),
                     jnp.zeros((16,), jnp.float32)))
                s_rows[rr, :] = s_rows[rr, :] + s_a
                g_rows[rr, :] = g_rows[rr, :] + g_a

        start(0, buf0, sem0)
        start(1, buf1, sem1)

        def ring(k, _):
            wait(buf0, sem0)
            proc(buf0, 2 * k)
            start(2 * k + 2, buf0, sem0)
            wait(buf1, sem1)
            proc(buf1, 2 * k + 1)
            start(2 * k + 3, buf1, sem1)
            return 0

        lax.fori_loop(0, (_NCH - 3) // 2, ring, 0)
        wait(buf0, sem0)
        proc(buf0, _NCH - 3)
        start(_NCH - 1, buf0, sem0)
        wait(buf1, sem1)
        proc(buf1, _NCH - 2)
        wait(buf0, sem0)
        proc(buf0, _NCH - 1)

        for rr in range(_RPW):
            t_spl = (tgt_v[...].at[jnp.full((16,), rr, jnp.int32)]
                     .get(mode="promise_in_bounds"))
            per_row = _EPS * s_rows[rr, :] + (_CONF - _EPS) * g_rows[rr, :]
            rows_v[rr, :] = per_row * jnp.minimum(t_spl, 1).astype(
                jnp.float32)
        pltpu.sync_copy(rows_v, out_hbm.at[pl.ds(wid * _RPW, _RPW)])

    return _sc_kernel


def kernel(logit, target):
    n = logit.shape[0]
    r_tc = n - _R_SC
    tgt = target.astype(jnp.int32)
    tail = jnp.pad(lax.slice(logit, (r_tc, _C_SC), (n, _C)),
                   ((0, 0), (0, _TAILW - (_C - _C_SC))))
    sc_out = _make_sc_kernel(r_tc)(logit, tgt)
    tc_out = _tc_part(logit, tgt.reshape(n, 1), tail,
                      tgt[r_tc:].reshape(_R_SC, 1), r_tc)
    return -(jnp.sum(tc_out) + jnp.sum(sc_out))


# speedup vs baseline: 1.0050x; 1.0037x over previous
import functools

import jax
import jax.numpy as jnp
from jax import lax
from jax.experimental import pallas as pl
from jax.experimental.pallas import tpu as pltpu
from jax.experimental.pallas import tpu_sc as plsc

_C = 100000
_IGNORE = 0
_SMOOTH = 0.1
_CONF = 1.0 - _SMOOTH
_EPS = _SMOOTH / (_C - 1)

_BR = 64
_R_SC = 256
_NC = 2
_NS = 16
_NW = _NC * _NS
_RPW = _R_SC // _NW
_CW = 2560
_C_SC = 99840
_NCH = _C_SC // _CW
_TAILW = 256
_UNROLL = 10


def _tc_body(tgt_ref, logit_ref, ttgt_ref, tail_ref, out_ref):
    i = pl.program_id(0)
    t = tgt_ref[...]
    col = jax.lax.broadcasted_iota(jnp.int32, (_BR, _C), 1)
    blk = logit_ref[...]
    s = jnp.sum(blk, axis=1, keepdims=True)
    g = jnp.sum(jnp.where(col == t, blk, 0.0), axis=1, keepdims=True)
    per_row = _EPS * s + (_CONF - _EPS) * g
    partial = jnp.sum(jnp.where(t != _IGNORE, per_row, 0.0))

    tt = ttgt_ref[...]
    tcol = jax.lax.broadcasted_iota(jnp.int32, (_R_SC, _TAILW), 1) + _C_SC
    tblk = tail_ref[...]
    ts = jnp.sum(tblk, axis=1, keepdims=True)
    tg = jnp.sum(jnp.where(tcol == tt, tblk, 0.0), axis=1, keepdims=True)
    tper = _EPS * ts + (_CONF - _EPS) * tg
    tpartial = jnp.sum(jnp.where(tt != _IGNORE, tper, 0.0))

    partial = partial + lax.select(i == 0, tpartial, 0.0)
    out_ref[...] = jnp.full((1, 1, 1), partial, jnp.float32)


def _tc_part(logit, tgt, tail, tail_tgt, r_tc):
    nblk = r_tc // _BR
    out = pl.pallas_call(
        _tc_body,
        grid=(nblk,),
        in_specs=[
            pl.BlockSpec((_BR, 1), lambda i: (i, 0)),
            pl.BlockSpec((_BR, _C), lambda i: (i, 0)),
            pl.BlockSpec((_R_SC, 1), lambda i: (0, 0)),
            pl.BlockSpec((_R_SC, _TAILW), lambda i: (0, 0)),
        ],
        out_specs=pl.BlockSpec((1, 1, 1), lambda i: (i, 0, 0)),
        out_shape=jax.ShapeDtypeStruct((nblk, 1, 1), jnp.float32),
        compiler_params=pltpu.CompilerParams(
            dimension_semantics=("arbitrary",),
        ),
    )(tgt, logit, tail_tgt, tail)
    return out


def _make_sc_kernel(r_tc):
    mesh = plsc.VectorSubcoreMesh(core_axis_name="c", subcore_axis_name="s")

    @functools.partial(
        pl.kernel,
        mesh=mesh,
        out_type=jax.ShapeDtypeStruct((_R_SC, 16), jnp.float32),
        scratch_types=[
            pltpu.VMEM((16,), jnp.int32),
            pltpu.VMEM((_RPW, _CW), jnp.float32),
            pltpu.VMEM((_RPW, _CW), jnp.float32),
            pltpu.VMEM((_RPW, 16), jnp.float32),
            pltpu.VMEM((_RPW, 16), jnp.float32),
            pltpu.VMEM((_RPW, 16), jnp.float32),
            pltpu.SemaphoreType.DMA,
            pltpu.SemaphoreType.DMA,
        ],
    )
    def _sc_kernel(logit_hbm, tgt_hbm, out_hbm, tgt_v, buf0, buf1,
                   s_rows, g_rows, rows_v, sem0, sem1):
        wid = lax.axis_index("s") * _NC + lax.axis_index("c")
        row_base = r_tc + wid * _RPW
        pltpu.sync_copy(tgt_hbm.at[pl.ds(row_base, _RPW)],
                        tgt_v.at[pl.ds(0, _RPW)])
        for rr in range(_RPW):
            s_rows[rr, :] = jnp.zeros((16,), jnp.float32)
            g_rows[rr, :] = jnp.zeros((16,), jnp.float32)

        def start(ch, buf, sem):
            pltpu.make_async_copy(
                logit_hbm.at[pl.ds(row_base, _RPW), pl.ds(ch * _CW, _CW)],
                buf, sem).start()

        def wait(buf, sem):
            pltpu.make_async_copy(
                logit_hbm.at[pl.ds(row_base, _RPW), pl.ds(0, _CW)],
                buf, sem).wait()

        def proc(buf, ch):
            base = ch * _CW
            for rr in range(_RPW):
                t_spl = (tgt_v[...].at[jnp.full((16,), rr, jnp.int32)]
                         .get(mode="promise_in_bounds"))

                def step(k, carry, rr=rr, t_spl=t_spl):
                    s_a, g_a = carry
                    for u in range(_UNROLL):
                        off = (k * _UNROLL + u) * 16
                        vals = buf[rr, pl.ds(off, 16)]
                        gcol = lax.iota(jnp.int32, 16) + (base + off)
                        s_a = s_a + vals
                        g_a = g_a + jnp.where(gcol == t_spl, vals, 0.0)
                    return s_a, g_a

                s_a, g_a = lax.fori_loop(
                    0, _CW // (16 * _UNROLL), step,
                    (jnp.zeros((16,), jnp.float32),
                     jnp.zeros((16,), jnp.float32)))
                s_rows[rr, :] = s_rows[rr, :] + s_a
                g_rows[rr, :] = g_rows[rr, :] + g_a

        start(0, buf0, sem0)
        start(1, buf1, sem1)

        def ring(k, _):
            wait(buf0, sem0)
            proc(buf0, 2 * k)
            start(2 * k + 2, buf0, sem0)
            wait(buf1, sem1)
            proc(buf1, 2 * k + 1)
            start(2 * k + 3, buf1, sem1)
            return 0

        lax.fori_loop(0, (_NCH - 3) // 2, ring, 0)
        wait(buf0, sem0)
        proc(buf0, _NCH - 3)
        start(_NCH - 1, buf0, sem0)
        wait(buf1, sem1)
        proc(buf1, _NCH - 2)
        wait(buf0, sem0)
        proc(buf0, _NCH - 1)

        for rr in range(_RPW):
            t_spl = (tgt_v[...].at[jnp.full((16,), rr, jnp.int32)]
                     .get(mode="promise_in_bounds"))
            per_row = _EPS * s_rows[rr, :] + (_CONF - _EPS) * g_rows[rr, :]
            rows_v[rr, :] = per_row * jnp.minimum(t_spl, 1).astype(
                jnp.float32)
        pltpu.sync_copy(rows_v, out_hbm.at[pl.ds(wid * _RPW, _RPW)])

    return _sc_kernel


def kernel(logit, target):
    n = logit.shape[0]
    r_tc = n - _R_SC
    tgt = target.astype(jnp.int32)
    tail = jnp.pad(lax.slice(logit, (r_tc, _C_SC), (n, _C)),
                   ((0, 0), (0, _TAILW - (_C - _C_SC))))
    sc_out = _make_sc_kernel(r_tc)(logit, tgt)
    tc_out = _tc_part(logit, tgt.reshape(n, 1), tail,
                      tgt[r_tc:].reshape(_R_SC, 1), r_tc)
    return -(jnp.sum(tc_out) + jnp.sum(sc_out))
